# Initial kernel scaffold; baseline (speedup 1.0000x reference)
import jax
import jax.numpy as jnp
from jax.experimental import pallas as pl


def kernel(feat_user, feat_item, edge_follows, edge_buys, W_emb_user, b_emb_user, W_emb_item, b_emb_item, W_follows, al_follows, ar_follows, W_buys, al_buys, ar_buys):
    raise NotImplementedError("write your pallas kernel here")


# speedup vs baseline: 1.6701x; 1.6701x over previous
import functools

import jax
import jax.numpy as jnp
from jax.experimental import pallas as pl
from jax.experimental.pallas import tpu as pltpu

H = 256
BLK = 1000


def _user_dense_body(feat, We, be, Wf, alf, arf, Wb, alb,
                     zf_o, zbs_o, elf_o, erf_o, elb_o):
    h = jnp.dot(feat[...], We[...], preferred_element_type=jnp.float32) + be[...]
    zf = jnp.dot(h, Wf[...], preferred_element_type=jnp.float32)
    zbs = jnp.dot(h, Wb[...], preferred_element_type=jnp.float32)
    zf_o[...] = zf
    zbs_o[...] = zbs
    elf_o[...] = jnp.dot(zf, alf[...], preferred_element_type=jnp.float32)
    erf_o[...] = jnp.dot(zf, arf[...], preferred_element_type=jnp.float32)
    elb_o[...] = jnp.dot(zbs, alb[...], preferred_element_type=jnp.float32)


def _item_dense_body(feat, We, be, Wb, arb, erb_o):
    h = jnp.dot(feat[...], We[...], preferred_element_type=jnp.float32) + be[...]
    w2 = jnp.dot(Wb[...], arb[...], preferred_element_type=jnp.float32)
    erb_o[...] = jnp.dot(h, w2, preferred_element_type=jnp.float32)


def _dense_user(feat_user, W_emb_user, b_emb_user, W_f, al_f, ar_f, W_b, al_b):
    n = feat_user.shape[0]
    grid = (n // BLK,)
    mat_spec = pl.BlockSpec((H, H), lambda i: (0, 0))
    vec_spec = pl.BlockSpec((H, 1), lambda i: (0, 0))
    row_spec = pl.BlockSpec((BLK, H), lambda i: (i, 0))
    col_spec = pl.BlockSpec((BLK, 1), lambda i: (i, 0))
    out_shapes = (
        jax.ShapeDtypeStruct((n, H), jnp.float32),
        jax.ShapeDtypeStruct((n, H), jnp.float32),
        jax.ShapeDtypeStruct((n, 1), jnp.float32),
        jax.ShapeDtypeStruct((n, 1), jnp.float32),
        jax.ShapeDtypeStruct((n, 1), jnp.float32),
    )
    return pl.pallas_call(
        _user_dense_body,
        grid=grid,
        in_specs=[row_spec, mat_spec, pl.BlockSpec((1, H), lambda i: (0, 0)),
                  mat_spec, vec_spec, vec_spec, mat_spec, vec_spec],
        out_specs=(row_spec, row_spec, col_spec, col_spec, col_spec),
        out_shape=out_shapes,
    )(feat_user, W_emb_user, b_emb_user.reshape(1, H), W_f,
      al_f.reshape(H, 1), ar_f.reshape(H, 1), W_b, al_b.reshape(H, 1))


def _dense_item(feat_item, W_emb_item, b_emb_item, W_b, ar_b):
    n = feat_item.shape[0]
    grid = (n // BLK,)
    mat_spec = pl.BlockSpec((H, H), lambda i: (0, 0))
    return pl.pallas_call(
        _item_dense_body,
        grid=grid,
        in_specs=[pl.BlockSpec((BLK, H), lambda i: (i, 0)), mat_spec,
                  pl.BlockSpec((1, H), lambda i: (0, 0)), mat_spec,
                  pl.BlockSpec((H, 1), lambda i: (0, 0))],
        out_specs=pl.BlockSpec((BLK, 1), lambda i: (i, 0)),
        out_shape=jax.ShapeDtypeStruct((n, 1), jnp.float32),
    )(feat_item, W_emb_item, b_emb_item.reshape(1, H), W_b, ar_b.reshape(H, 1))


def _edge_phase_jax(z_src, el, er, src, dst, num_dst):
    e = jax.nn.leaky_relu(el[src] + er[dst], negative_slope=0.2)
    ex = jnp.exp(e)
    denom = jax.ops.segment_sum(ex, dst, num_segments=num_dst)
    num = jax.ops.segment_sum(ex[:, None] * z_src[src], dst, num_segments=num_dst)
    return jax.nn.relu(num / (denom[:, None] + 1e-9))


def kernel(feat_user, feat_item, edge_follows, edge_buys,
           W_emb_user, b_emb_user, W_emb_item, b_emb_item,
           W_follows, al_follows, ar_follows, W_buys, al_buys, ar_buys):
    edge_follows = edge_follows.astype(jnp.int32)
    edge_buys = edge_buys.astype(jnp.int32)
    z_f, z_bs, el_f, er_f, el_b = _dense_user(
        feat_user, W_emb_user, b_emb_user,
        W_follows, al_follows, ar_follows, W_buys, al_buys)
    er_b = _dense_item(feat_item, W_emb_item, b_emb_item, W_buys, ar_buys)

    n_user = feat_user.shape[0]
    n_item = feat_item.shape[0]
    out_user = _edge_phase_jax(z_f, el_f[:, 0], er_f[:, 0],
                               edge_follows[0], edge_follows[1], n_user)
    out_item = _edge_phase_jax(z_bs, el_b[:, 0], er_b[:, 0],
                               edge_buys[0], edge_buys[1], n_item)
    return out_user, out_item


# speedup vs baseline: 7.9605x; 4.7665x over previous
import functools

import jax
import jax.numpy as jnp
from jax import lax
from jax.experimental import pallas as pl
from jax.experimental.pallas import tpu as pltpu
from jax.experimental.pallas import tpu_sc as plsc

H = 256
HB = 128
HW = 144
BLK = 1000
N = 10000
E = 160000
NC = 2
NS = 16
HALF = N // NC
HALFP = 5120
EPT = E // NS
BATCH = 80
NB = EPT // BATCH
STRIPE = HALFP // NS


def _user_dense_body(feat, We, be, Wf, alf, arf, Wb, alb,
                     zf0_o, zf1_o, zb0_o, zb1_o, elf_o, erf_o, elb_o):
    h = jnp.dot(feat[...], We[...], preferred_element_type=jnp.float32) + be[...]
    zf = jnp.dot(h, Wf[...], preferred_element_type=jnp.float32)
    zbs = jnp.dot(h, Wb[...], preferred_element_type=jnp.float32)
    ones = jnp.ones((BLK, HW - HB), jnp.float32)
    zf0_o[...] = jnp.concatenate([zf[:, :HB], ones], axis=1)
    zf1_o[...] = jnp.concatenate([zf[:, HB:], ones], axis=1)
    zb0_o[...] = jnp.concatenate([zbs[:, :HB], ones], axis=1)
    zb1_o[...] = jnp.concatenate([zbs[:, HB:], ones], axis=1)
    elf_o[...] = jnp.dot(zf, alf[...], preferred_element_type=jnp.float32)
    erf_o[...] = jnp.dot(zf, arf[...], preferred_element_type=jnp.float32)
    elb_o[...] = jnp.dot(zbs, alb[...], preferred_element_type=jnp.float32)


def _item_dense_body(feat, We, be, Wb, arb, erb_o):
    h = jnp.dot(feat[...], We[...], preferred_element_type=jnp.float32) + be[...]
    w2 = jnp.dot(Wb[...], arb[...], preferred_element_type=jnp.float32)
    erb_o[...] = jnp.dot(h, w2, preferred_element_type=jnp.float32)


def _dense_user(feat_user, W_emb_user, b_emb_user, W_f, al_f, ar_f, W_b, al_b):
    grid = (N // BLK,)
    mat_spec = pl.BlockSpec((H, H), lambda i: (0, 0))
    vec_spec = pl.BlockSpec((H, 1), lambda i: (0, 0))
    z_spec = pl.BlockSpec((BLK, HW), lambda i: (i, 0))
    col_spec = pl.BlockSpec((BLK, 1), lambda i: (i, 0))
    zs = jax.ShapeDtypeStruct((N, HW), jnp.float32)
    cs = jax.ShapeDtypeStruct((N, 1), jnp.float32)
    return pl.pallas_call(
        _user_dense_body,
        grid=grid,
        in_specs=[pl.BlockSpec((BLK, H), lambda i: (i, 0)), mat_spec,
                  pl.BlockSpec((1, H), lambda i: (0, 0)),
                  mat_spec, vec_spec, vec_spec, mat_spec, vec_spec],
        out_specs=(z_spec, z_spec, z_spec, z_spec, col_spec, col_spec, col_spec),
        out_shape=(zs, zs, zs, zs, cs, cs, cs),
    )(feat_user, W_emb_user, b_emb_user.reshape(1, H), W_f,
      al_f.reshape(H, 1), ar_f.reshape(H, 1), W_b, al_b.reshape(H, 1))


def _dense_item(feat_item, W_emb_item, b_emb_item, W_b, ar_b):
    grid = (N // BLK,)
    mat_spec = pl.BlockSpec((H, H), lambda i: (0, 0))
    return pl.pallas_call(
        _item_dense_body,
        grid=grid,
        in_specs=[pl.BlockSpec((BLK, H), lambda i: (i, 0)), mat_spec,
                  pl.BlockSpec((1, H), lambda i: (0, 0)), mat_spec,
                  pl.BlockSpec((H, 1), lambda i: (0, 0))],
        out_specs=pl.BlockSpec((BLK, 1), lambda i: (i, 0)),
        out_shape=jax.ShapeDtypeStruct((N, 1), jnp.float32),
    )(feat_item, W_emb_item, b_emb_item.reshape(1, H), W_b, ar_b.reshape(H, 1))


_MESH = plsc.VectorSubcoreMesh(core_axis_name="c", subcore_axis_name="s")


@functools.partial(
    pl.kernel, mesh=_MESH,
    compiler_params=pltpu.CompilerParams(needs_layout_passes=False,
                                         use_tc_tiling_on_sc=False),
    out_type=(jax.ShapeDtypeStruct((N, HB), jnp.float32),
              jax.ShapeDtypeStruct((N, HB), jnp.float32)),
    scratch_types=[
        pltpu.VMEM((N,), jnp.float32),
        pltpu.VMEM((HALF,), jnp.float32),
        pltpu.VMEM((EPT,), jnp.int32),
        pltpu.VMEM((NB, BATCH), jnp.int32),
        pltpu.VMEM((EPT,), jnp.float32),
        pltpu.VMEM((BATCH, HW), jnp.float32),
        pltpu.VMEM((BATCH, HW), jnp.float32),
        pltpu.VMEM((BATCH, HB), jnp.float32),
        pltpu.VMEM_SHARED((HALFP, HW), jnp.float32),
        pltpu.SemaphoreType.DMA,
    ],
)
def _edge_sc(z0_hbm, z1_hbm, el_hbm, er_hbm, src_hbm, dst3_hbm,
             out0_hbm, out1_hbm,
             el_v, er_v, src_v, dst2d, ex_v, rowbuf, zbuf, outbuf, num_sh, sem):
    c = lax.axis_index("c")
    s = lax.axis_index("s")

    pltpu.sync_copy(el_hbm, el_v)
    pltpu.sync_copy(er_hbm.at[pl.ds(c * HALF, HALF)], er_v)
    pltpu.sync_copy(src_hbm.at[pl.ds(s * EPT, EPT)], src_v)
    pltpu.sync_copy(dst3_hbm.at[s], dst2d)

    def _zrow(r, _):
        for v in range(HW // 16):
            zbuf[r, pl.ds(v * 16, 16)] = jnp.zeros((16,), jnp.float32)
        return 0
    lax.fori_loop(0, BATCH, _zrow, 0)

    def _phA(j, _):
        for u in range(BATCH // 16):
            sl = pl.ds(j * BATCH + u * 16, 16)
            slu = pl.ds(u * 16, 16)
            srcv = src_v[sl]
            dl = dst2d[j, slu] - c * HALF
            own = (dl >= 0) & (dl < HALF)
            elg = plsc.load_gather(el_v, [srcv])
            erg = plsc.load_gather(er_v, [jnp.where(own, dl, 0)])
            e = elg + erg
            e = jnp.where(e > 0, e, 0.2 * e)
            ex_v[sl] = jnp.where(own, jnp.exp(e), 0.0)
            dst2d[j, slu] = jnp.where(own, dl, HALF + (srcv & 63))
        return 0
    lax.fori_loop(0, NB, _phA, 0)

    for p, (zp, outp) in enumerate(((z0_hbm, out0_hbm), (z1_hbm, out1_hbm))):
        for b in range(STRIPE // BATCH):
            pltpu.sync_copy(zbuf, num_sh.at[pl.ds(s * STRIPE + b * BATCH, BATCH)])
        plsc.subcore_barrier()

        def _phB(j, _):
            pltpu.async_copy(zp.at[src_v.at[pl.ds(j * BATCH, BATCH)]],
                             rowbuf, sem).wait()

            def _scale(g, _):
                exv = ex_v[pl.ds(j * BATCH + g * 16, 16)]
                for k in range(16):
                    exs = exv[k]
                    r = g * 16 + k
                    for v in range(HW // 16):
                        sl2 = pl.ds(v * 16, 16)
                        rowbuf[r, sl2] = rowbuf[r, sl2] * exs
                return 0
            lax.fori_loop(0, BATCH // 16, _scale, 0)
            pltpu.sync_copy(rowbuf, num_sh.at[dst2d.at[j]], add=True)
            return 0
        lax.fori_loop(0, NB, _phB, 0)
        plsc.subcore_barrier()

        base = s * STRIPE
        for b in range(STRIPE // BATCH):
            row0 = base + b * BATCH

            @pl.when(row0 < HALF)
            def _fin():
                pltpu.sync_copy(num_sh.at[pl.ds(row0, BATCH)], rowbuf)

                def _grp(g, _):
                    ridx = lax.iota(jnp.int32, 16) + g * 16
                    cidx = jnp.full((16,), HB, jnp.int32)
                    den = plsc.load_gather(rowbuf, [ridx, cidx])
                    invv = jnp.ones((16,), jnp.float32) / (den + 1e-9)
                    for k in range(16):
                        inv = invv[k]
                        r = g * 16 + k
                        for v in range(HB // 16):
                            sl2 = pl.ds(v * 16, 16)
                            outbuf[r, sl2] = jnp.maximum(rowbuf[r, sl2] * inv, 0.0)
                    return 0
                lax.fori_loop(0, BATCH // 16, _grp, 0)

                g0 = c * HALF + row0

                @pl.when(row0 + BATCH <= HALF)
                def _full():
                    pltpu.sync_copy(outbuf, outp.at[pl.ds(g0, BATCH)])

                @pl.when(row0 + BATCH > HALF)
                def _part():
                    rem = HALF % BATCH
                    pltpu.sync_copy(outbuf.at[pl.ds(0, rem)],
                                    outp.at[pl.ds(g0, rem)])


def kernel(feat_user, feat_item, edge_follows, edge_buys,
           W_emb_user, b_emb_user, W_emb_item, b_emb_item,
           W_follows, al_follows, ar_follows, W_buys, al_buys, ar_buys):
    edge_follows = edge_follows.astype(jnp.int32)
    edge_buys = edge_buys.astype(jnp.int32)
    zf0, zf1, zb0, zb1, el_f, er_f, el_b = _dense_user(
        feat_user, W_emb_user, b_emb_user,
        W_follows, al_follows, ar_follows, W_buys, al_buys)
    er_b = _dense_item(feat_item, W_emb_item, b_emb_item, W_buys, ar_buys)

    dst_f = edge_follows[1].reshape(NS, NB, BATCH)
    dst_b = edge_buys[1].reshape(NS, NB, BATCH)
    u0, u1 = _edge_sc(zf0, zf1, el_f[:, 0], er_f[:, 0], edge_follows[0], dst_f)
    i0, i1 = _edge_sc(zb0, zb1, el_b[:, 0], er_b[:, 0], edge_buys[0], dst_b)
    out_user = jnp.concatenate([u0, u1], axis=1)
    out_item = jnp.concatenate([i0, i1], axis=1)
    return out_user, out_item


# speedup vs baseline: 9.2704x; 1.1645x over previous
import functools

import jax
import jax.numpy as jnp
from jax import lax
from jax.experimental import pallas as pl
from jax.experimental.pallas import tpu as pltpu
from jax.experimental.pallas import tpu_sc as plsc

H = 256
QB = 64
QW = 80
NQ = H // QB
BLK = 1000
N = 10000
E = 160000
NC = 2
NS = 16
NP = 10240
EPT = E // NS
BATCH = 80
NB = EPT // BATCH
STRIPE = NP // NS


def _user_dense_body(feat, We, be, Wf, alf, arf, Wb, alb, *outs):
    h = jnp.dot(feat[...], We[...], preferred_element_type=jnp.float32) + be[...]
    zf = jnp.dot(h, Wf[...], preferred_element_type=jnp.float32)
    zbs = jnp.dot(h, Wb[...], preferred_element_type=jnp.float32)
    ones = jnp.ones((BLK, QW - QB), jnp.float32)
    for p in range(NQ):
        outs[p][...] = jnp.concatenate([zf[:, p * QB:(p + 1) * QB], ones], axis=1)
        outs[NQ + p][...] = jnp.concatenate(
            [zbs[:, p * QB:(p + 1) * QB], ones], axis=1)
    outs[2 * NQ][...] = jnp.dot(zf, alf[...], preferred_element_type=jnp.float32)
    outs[2 * NQ + 1][...] = jnp.dot(zf, arf[...],
                                    preferred_element_type=jnp.float32)
    outs[2 * NQ + 2][...] = jnp.dot(zbs, alb[...],
                                    preferred_element_type=jnp.float32)


def _item_dense_body(feat, We, be, Wb, arb, erb_o):
    h = jnp.dot(feat[...], We[...], preferred_element_type=jnp.float32) + be[...]
    w2 = jnp.dot(Wb[...], arb[...], preferred_element_type=jnp.float32)
    erb_o[...] = jnp.dot(h, w2, preferred_element_type=jnp.float32)


def _dense_user(feat_user, W_emb_user, b_emb_user, W_f, al_f, ar_f, W_b, al_b):
    grid = (N // BLK,)
    mat_spec = pl.BlockSpec((H, H), lambda i: (0, 0))
    vec_spec = pl.BlockSpec((H, 1), lambda i: (0, 0))
    z_spec = pl.BlockSpec((BLK, QW), lambda i: (i, 0))
    col_spec = pl.BlockSpec((BLK, 1), lambda i: (i, 0))
    zs = jax.ShapeDtypeStruct((N, QW), jnp.float32)
    cs = jax.ShapeDtypeStruct((N, 1), jnp.float32)
    return pl.pallas_call(
        _user_dense_body,
        grid=grid,
        in_specs=[pl.BlockSpec((BLK, H), lambda i: (i, 0)), mat_spec,
                  pl.BlockSpec((1, H), lambda i: (0, 0)),
                  mat_spec, vec_spec, vec_spec, mat_spec, vec_spec],
        out_specs=(z_spec,) * (2 * NQ) + (col_spec,) * 3,
        out_shape=(zs,) * (2 * NQ) + (cs,) * 3,
    )(feat_user, W_emb_user, b_emb_user.reshape(1, H), W_f,
      al_f.reshape(H, 1), ar_f.reshape(H, 1), W_b, al_b.reshape(H, 1))


def _dense_item(feat_item, W_emb_item, b_emb_item, W_b, ar_b):
    grid = (N // BLK,)
    mat_spec = pl.BlockSpec((H, H), lambda i: (0, 0))
    return pl.pallas_call(
        _item_dense_body,
        grid=grid,
        in_specs=[pl.BlockSpec((BLK, H), lambda i: (i, 0)), mat_spec,
                  pl.BlockSpec((1, H), lambda i: (0, 0)), mat_spec,
                  pl.BlockSpec((H, 1), lambda i: (0, 0))],
        out_specs=pl.BlockSpec((BLK, 1), lambda i: (i, 0)),
        out_shape=jax.ShapeDtypeStruct((N, 1), jnp.float32),
    )(feat_item, W_emb_item, b_emb_item.reshape(1, H), W_b, ar_b.reshape(H, 1))


_MESH = plsc.VectorSubcoreMesh(core_axis_name="c", subcore_axis_name="s")


@functools.partial(
    pl.kernel, mesh=_MESH,
    compiler_params=pltpu.CompilerParams(needs_layout_passes=False,
                                         use_tc_tiling_on_sc=False),
    out_type=jax.ShapeDtypeStruct((NQ * 2 * N, QB), jnp.float32),
    scratch_types=[
        pltpu.VMEM((N,), jnp.float32),
        pltpu.VMEM((N,), jnp.float32),
        pltpu.VMEM((EPT,), jnp.int32),
        pltpu.VMEM((EPT,), jnp.int32),
        pltpu.VMEM((NB, BATCH), jnp.int32),
        pltpu.VMEM((EPT,), jnp.float32),
        pltpu.VMEM((BATCH, QW), jnp.float32),
        pltpu.VMEM((BATCH, QW), jnp.float32),
        pltpu.VMEM((BATCH, QB), jnp.float32),
        pltpu.VMEM_SHARED((NP, QW), jnp.float32),
        pltpu.SemaphoreType.DMA,
    ],
)
def _edge_sc(z_hbm, el_hbm, er_hbm, src_hbm, dst3_hbm, out_hbm,
             el_v, er_v, src_v, srcp_v, dst2d, ex_v, rowbuf, zbuf, outbuf,
             num_sh, sem):
    c = lax.axis_index("c")
    s = lax.axis_index("s")

    pltpu.sync_copy(el_hbm.at[pl.ds(c * N, N)], el_v)
    pltpu.sync_copy(er_hbm.at[pl.ds(c * N, N)], er_v)
    pltpu.sync_copy(src_hbm.at[pl.ds(c * E + s * EPT, EPT)], src_v)
    pltpu.sync_copy(dst3_hbm.at[c * NS + s], dst2d)

    def _zrow(r, _):
        for v in range(QW // 16):
            zbuf[r, pl.ds(v * 16, 16)] = jnp.zeros((16,), jnp.float32)
        return 0
    lax.fori_loop(0, BATCH, _zrow, 0)

    def _phA(j, _):
        for u in range(BATCH // 16):
            sl = pl.ds(j * BATCH + u * 16, 16)
            srcv = src_v[sl] - c * N
            dv = dst2d[j, pl.ds(u * 16, 16)]
            e = plsc.load_gather(el_v, [srcv]) + plsc.load_gather(er_v, [dv])
            e = jnp.where(e > 0, e, 0.2 * e)
            ex_v[sl] = jnp.exp(e)
        return 0
    lax.fori_loop(0, NB, _phA, 0)

    def _pass(p, _):
        def _off(i, _):
            sl = pl.ds(i * 16, 16)
            srcp_v[sl] = src_v[sl] + p * (2 * N)
            return 0
        lax.fori_loop(0, EPT // 16, _off, 0)

        def _zero(b, _):
            pltpu.sync_copy(zbuf, num_sh.at[pl.ds(s * STRIPE + b * BATCH, BATCH)])
            return 0
        lax.fori_loop(0, STRIPE // BATCH, _zero, 0)
        plsc.subcore_barrier()

        def _phB(j, _):
            pltpu.async_copy(z_hbm.at[srcp_v.at[pl.ds(j * BATCH, BATCH)]],
                             rowbuf, sem).wait()

            def _scale(g, _):
                exv = ex_v[pl.ds(j * BATCH + g * 16, 16)]
                for k in range(16):
                    exs = exv[k]
                    r = g * 16 + k
                    for v in range(QW // 16):
                        sl2 = pl.ds(v * 16, 16)
                        rowbuf[r, sl2] = rowbuf[r, sl2] * exs
                return 0
            lax.fori_loop(0, BATCH // 16, _scale, 0)
            pltpu.sync_copy(rowbuf, num_sh.at[dst2d.at[j]], add=True)
            return 0
        lax.fori_loop(0, NB, _phB, 0)
        plsc.subcore_barrier()

        def _fin(b, _):
            row0 = s * STRIPE + b * BATCH

            @pl.when(row0 < N)
            def _():
                pltpu.sync_copy(num_sh.at[pl.ds(row0, BATCH)], rowbuf)

                def _grp(g, _):
                    ridx = lax.iota(jnp.int32, 16) + g * 16
                    cidx = jnp.full((16,), QB, jnp.int32)
                    den = plsc.load_gather(rowbuf, [ridx, cidx])
                    invv = jnp.ones((16,), jnp.float32) / (den + 1e-9)
                    for k in range(16):
                        inv = invv[k]
                        r = g * 16 + k
                        for v in range(QB // 16):
                            sl2 = pl.ds(v * 16, 16)
                            outbuf[r, sl2] = jnp.maximum(
                                rowbuf[r, sl2] * inv, 0.0)
                    return 0
                lax.fori_loop(0, BATCH // 16, _grp, 0)
                pltpu.sync_copy(
                    outbuf, out_hbm.at[pl.ds(p * 2 * N + c * N + row0, BATCH)])
            return 0
        lax.fori_loop(0, STRIPE // BATCH, _fin, 0)
        return 0
    lax.fori_loop(0, NQ, _pass, 0)


def kernel(feat_user, feat_item, edge_follows, edge_buys,
           W_emb_user, b_emb_user, W_emb_item, b_emb_item,
           W_follows, al_follows, ar_follows, W_buys, al_buys, ar_buys):
    edge_follows = edge_follows.astype(jnp.int32)
    edge_buys = edge_buys.astype(jnp.int32)
    du = _dense_user(feat_user, W_emb_user, b_emb_user,
                     W_follows, al_follows, ar_follows, W_buys, al_buys)
    qf = du[:NQ]
    qb = du[NQ:2 * NQ]
    el_f, er_f, el_b = du[2 * NQ], du[2 * NQ + 1], du[2 * NQ + 2]
    er_b = _dense_item(feat_item, W_emb_item, b_emb_item, W_buys, ar_buys)

    z_all = jnp.concatenate(
        [blk for p in range(NQ) for blk in (qf[p], qb[p])], axis=0)
    el_all = jnp.concatenate([el_f[:, 0], el_b[:, 0]])
    er_all = jnp.concatenate([er_f[:, 0], er_b[:, 0]])
    src_all = jnp.concatenate([edge_follows[0], edge_buys[0] + N])
    dst3 = jnp.concatenate(
        [edge_follows[1].reshape(NS, NB, BATCH),
         edge_buys[1].reshape(NS, NB, BATCH)], axis=0)

    out = _edge_sc(z_all, el_all, er_all, src_all, dst3)
    out_user = jnp.concatenate(
        [out[p * 2 * N:p * 2 * N + N] for p in range(NQ)], axis=1)
    out_item = jnp.concatenate(
        [out[p * 2 * N + N:(p + 1) * 2 * N] for p in range(NQ)], axis=1)
    return out_user, out_item


# speedup vs baseline: 11.2706x; 1.2158x over previous
import functools

import jax
import jax.numpy as jnp
from jax import lax
from jax.experimental import pallas as pl
from jax.experimental.pallas import tpu as pltpu
from jax.experimental.pallas import tpu_sc as plsc

H = 256
QB = 64
QW = 80
NQ = H // QB
BLK = 1000
N = 10000
E = 160000
NC = 2
NS = 16
NP = 10240
EPT = E // NS
BATCH = 80
NB = EPT // BATCH
STRIPE = NP // NS


def _user_dense_body(feat, We, be, Wf, alf, arf, Wb, alb, *outs):
    h = jnp.dot(feat[...], We[...], preferred_element_type=jnp.float32) + be[...]
    zf = jnp.dot(h, Wf[...], preferred_element_type=jnp.float32)
    zbs = jnp.dot(h, Wb[...], preferred_element_type=jnp.float32)
    ones = jnp.ones((BLK, QW - QB), jnp.float32)
    for p in range(NQ):
        outs[p][...] = jnp.concatenate([zf[:, p * QB:(p + 1) * QB], ones], axis=1)
        outs[NQ + p][...] = jnp.concatenate(
            [zbs[:, p * QB:(p + 1) * QB], ones], axis=1)
    outs[2 * NQ][...] = jnp.dot(zf, alf[...], preferred_element_type=jnp.float32)
    outs[2 * NQ + 1][...] = jnp.dot(zf, arf[...],
                                    preferred_element_type=jnp.float32)
    outs[2 * NQ + 2][...] = jnp.dot(zbs, alb[...],
                                    preferred_element_type=jnp.float32)


def _item_dense_body(feat, We, be, Wb, arb, erb_o):
    h = jnp.dot(feat[...], We[...], preferred_element_type=jnp.float32) + be[...]
    w2 = jnp.dot(Wb[...], arb[...], preferred_element_type=jnp.float32)
    erb_o[...] = jnp.dot(h, w2, preferred_element_type=jnp.float32)


def _dense_user(feat_user, W_emb_user, b_emb_user, W_f, al_f, ar_f, W_b, al_b):
    grid = (N // BLK,)
    mat_spec = pl.BlockSpec((H, H), lambda i: (0, 0))
    vec_spec = pl.BlockSpec((H, 1), lambda i: (0, 0))
    z_spec = pl.BlockSpec((BLK, QW), lambda i: (i, 0))
    col_spec = pl.BlockSpec((BLK, 1), lambda i: (i, 0))
    zs = jax.ShapeDtypeStruct((N, QW), jnp.float32)
    cs = jax.ShapeDtypeStruct((N, 1), jnp.float32)
    return pl.pallas_call(
        _user_dense_body,
        grid=grid,
        in_specs=[pl.BlockSpec((BLK, H), lambda i: (i, 0)), mat_spec,
                  pl.BlockSpec((1, H), lambda i: (0, 0)),
                  mat_spec, vec_spec, vec_spec, mat_spec, vec_spec],
        out_specs=(z_spec,) * (2 * NQ) + (col_spec,) * 3,
        out_shape=(zs,) * (2 * NQ) + (cs,) * 3,
    )(feat_user, W_emb_user, b_emb_user.reshape(1, H), W_f,
      al_f.reshape(H, 1), ar_f.reshape(H, 1), W_b, al_b.reshape(H, 1))


def _dense_item(feat_item, W_emb_item, b_emb_item, W_b, ar_b):
    grid = (N // BLK,)
    mat_spec = pl.BlockSpec((H, H), lambda i: (0, 0))
    return pl.pallas_call(
        _item_dense_body,
        grid=grid,
        in_specs=[pl.BlockSpec((BLK, H), lambda i: (i, 0)), mat_spec,
                  pl.BlockSpec((1, H), lambda i: (0, 0)), mat_spec,
                  pl.BlockSpec((H, 1), lambda i: (0, 0))],
        out_specs=pl.BlockSpec((BLK, 1), lambda i: (i, 0)),
        out_shape=jax.ShapeDtypeStruct((N, 1), jnp.float32),
    )(feat_item, W_emb_item, b_emb_item.reshape(1, H), W_b, ar_b.reshape(H, 1))


_MESH = plsc.VectorSubcoreMesh(core_axis_name="c", subcore_axis_name="s")


@functools.partial(
    pl.kernel, mesh=_MESH,
    compiler_params=pltpu.CompilerParams(needs_layout_passes=False,
                                         use_tc_tiling_on_sc=False),
    out_type=jax.ShapeDtypeStruct((NQ * 2 * N, QB), jnp.float32),
    scratch_types=[
        pltpu.VMEM((N,), jnp.float32),
        pltpu.VMEM((N,), jnp.float32),
        pltpu.VMEM((EPT,), jnp.int32),
        pltpu.VMEM((NB, BATCH), jnp.int32),
        pltpu.VMEM((EPT,), jnp.float32),
        pltpu.VMEM((BATCH, QW), jnp.float32),
        pltpu.VMEM((BATCH, QW), jnp.float32),
        pltpu.VMEM((BATCH, QW), jnp.float32),
        pltpu.VMEM((BATCH, QB), jnp.float32),
        pltpu.VMEM_SHARED((NP, QW), jnp.float32),
        pltpu.SemaphoreType.DMA,
        pltpu.SemaphoreType.DMA,
    ],
)
def _edge_sc(z_hbm, el_hbm, er_hbm, src_hbm, dst3_hbm, out_hbm,
             el_v, er_v, srcp_v, dst2d, ex_v, rowbuf, rowbuf2, zbuf,
             outbuf, num_sh, sem, sem2):
    c = lax.axis_index("c")
    s = lax.axis_index("s")

    pltpu.sync_copy(el_hbm.at[pl.ds(c * N, N)], el_v)
    pltpu.sync_copy(er_hbm.at[pl.ds(c * N, N)], er_v)
    pltpu.sync_copy(src_hbm.at[pl.ds(c * E + s * EPT, EPT)], srcp_v)
    pltpu.sync_copy(dst3_hbm.at[c * NS + s], dst2d)

    def _zrow(r, _):
        for v in range(QW // 16):
            zbuf[r, pl.ds(v * 16, 16)] = jnp.zeros((16,), jnp.float32)
        return 0
    lax.fori_loop(0, BATCH, _zrow, 0)

    def _phA(j, _):
        for u in range(BATCH // 16):
            sl = pl.ds(j * BATCH + u * 16, 16)
            srcv = srcp_v[sl] - c * N
            dv = dst2d[j, pl.ds(u * 16, 16)]
            e = plsc.load_gather(el_v, [srcv]) + plsc.load_gather(er_v, [dv])
            e = jnp.where(e > 0, e, 0.2 * e)
            ex_v[sl] = jnp.exp(e)
        return 0
    lax.fori_loop(0, NB, _phA, 0)

    def _pass(p, _):
        @pl.when(p > 0)
        def _advance():
            def _off(i, _):
                sl = pl.ds(i * 16, 16)
                srcp_v[sl] = srcp_v[sl] + 2 * N
                return 0
            lax.fori_loop(0, EPT // 16, _off, 0)

        def _zero(b, _):
            pltpu.sync_copy(zbuf, num_sh.at[pl.ds(s * STRIPE + b * BATCH, BATCH)])
            return 0
        lax.fori_loop(0, STRIPE // BATCH, _zero, 0)
        plsc.subcore_barrier()

        def _gather(j, buf, sm):
            return pltpu.async_copy(z_hbm.at[srcp_v.at[pl.ds(j * BATCH, BATCH)]],
                                    buf, sm)

        def _proc(j, buf):
            def _scale(g, _):
                exv = ex_v[pl.ds(j * BATCH + g * 16, 16)]
                for k in range(16):
                    exs = exv[k]
                    r = g * 16 + k
                    for v in range(QW // 16):
                        sl2 = pl.ds(v * 16, 16)
                        buf[r, sl2] = buf[r, sl2] * exs
                return 0
            lax.fori_loop(0, BATCH // 16, _scale, 0)
            pltpu.sync_copy(buf, num_sh.at[dst2d.at[j]], add=True)

        def _phB(j2, _):
            j = 2 * j2
            ha = _gather(j, rowbuf, sem)
            hb = _gather(j + 1, rowbuf2, sem2)
            ha.wait()
            _proc(j, rowbuf)
            hb.wait()
            _proc(j + 1, rowbuf2)
            return 0
        lax.fori_loop(0, NB // 2, _phB, 0)
        _gather(NB - 1, rowbuf, sem).wait()
        _proc(NB - 1, rowbuf)
        plsc.subcore_barrier()

        def _fin(b, _):
            row0 = s * STRIPE + b * BATCH

            @pl.when(row0 < N)
            def _():
                pltpu.sync_copy(num_sh.at[pl.ds(row0, BATCH)], rowbuf)

                def _grp(g, _):
                    ridx = lax.iota(jnp.int32, 16) + g * 16
                    cidx = jnp.full((16,), QB, jnp.int32)
                    den = plsc.load_gather(rowbuf, [ridx, cidx])
                    invv = jnp.ones((16,), jnp.float32) / (den + 1e-9)
                    for k in range(16):
                        inv = invv[k]
                        r = g * 16 + k
                        for v in range(QB // 16):
                            sl2 = pl.ds(v * 16, 16)
                            outbuf[r, sl2] = jnp.maximum(
                                rowbuf[r, sl2] * inv, 0.0)
                    return 0
                lax.fori_loop(0, BATCH // 16, _grp, 0)
                pltpu.sync_copy(
                    outbuf, out_hbm.at[pl.ds(p * 2 * N + c * N + row0, BATCH)])
            return 0
        lax.fori_loop(0, STRIPE // BATCH, _fin, 0)
        return 0
    lax.fori_loop(0, NQ, _pass, 0)


def kernel(feat_user, feat_item, edge_follows, edge_buys,
           W_emb_user, b_emb_user, W_emb_item, b_emb_item,
           W_follows, al_follows, ar_follows, W_buys, al_buys, ar_buys):
    edge_follows = edge_follows.astype(jnp.int32)
    edge_buys = edge_buys.astype(jnp.int32)
    du = _dense_user(feat_user, W_emb_user, b_emb_user,
                     W_follows, al_follows, ar_follows, W_buys, al_buys)
    qf = du[:NQ]
    qb = du[NQ:2 * NQ]
    el_f, er_f, el_b = du[2 * NQ], du[2 * NQ + 1], du[2 * NQ + 2]
    er_b = _dense_item(feat_item, W_emb_item, b_emb_item, W_buys, ar_buys)

    z_all = jnp.concatenate(
        [blk for p in range(NQ) for blk in (qf[p], qb[p])], axis=0)
    el_all = jnp.concatenate([el_f[:, 0], el_b[:, 0]])
    er_all = jnp.concatenate([er_f[:, 0], er_b[:, 0]])
    src_all = jnp.concatenate([edge_follows[0], edge_buys[0] + N])
    dst3 = jnp.concatenate(
        [edge_follows[1].reshape(NS, NB, BATCH),
         edge_buys[1].reshape(NS, NB, BATCH)], axis=0)

    out = _edge_sc(z_all, el_all, er_all, src_all, dst3)
    out_user = jnp.concatenate(
        [out[p * 2 * N:p * 2 * N + N] for p in range(NQ)], axis=1)
    out_item = jnp.concatenate(
        [out[p * 2 * N + N:(p + 1) * 2 * N] for p in range(NQ)], axis=1)
    return out_user, out_item


# speedup vs baseline: 12.3572x; 1.0964x over previous
import functools

import jax
import jax.numpy as jnp
from jax import lax
from jax.experimental import pallas as pl
from jax.experimental.pallas import tpu as pltpu
from jax.experimental.pallas import tpu_sc as plsc

H = 256
QB = 64
QW = 80
NQ = H // QB
BLK = 1000
N = 10000
E = 160000
NC = 2
NS = 16
NP = 10240
EPT = E // NS
BATCH = 80
NB = EPT // BATCH
STRIPE = NP // NS


def _user_dense_body(feat, We, be, Wf, alf, arf, Wb, alb, *outs):
    h = jnp.dot(feat[...], We[...], preferred_element_type=jnp.float32) + be[...]
    zf = jnp.dot(h, Wf[...], preferred_element_type=jnp.float32)
    zbs = jnp.dot(h, Wb[...], preferred_element_type=jnp.float32)
    ones = jnp.ones((BLK, QW - QB), jnp.float32)
    for p in range(NQ):
        outs[p][...] = jnp.concatenate([zf[:, p * QB:(p + 1) * QB], ones], axis=1)
        outs[NQ + p][...] = jnp.concatenate(
            [zbs[:, p * QB:(p + 1) * QB], ones], axis=1)
    outs[2 * NQ][...] = jnp.dot(zf, alf[...], preferred_element_type=jnp.float32)
    outs[2 * NQ + 1][...] = jnp.dot(zf, arf[...],
                                    preferred_element_type=jnp.float32)
    outs[2 * NQ + 2][...] = jnp.dot(zbs, alb[...],
                                    preferred_element_type=jnp.float32)


def _item_dense_body(feat, We, be, Wb, arb, erb_o):
    h = jnp.dot(feat[...], We[...], preferred_element_type=jnp.float32) + be[...]
    w2 = jnp.dot(Wb[...], arb[...], preferred_element_type=jnp.float32)
    erb_o[...] = jnp.dot(h, w2, preferred_element_type=jnp.float32)


def _dense_user(feat_user, W_emb_user, b_emb_user, W_f, al_f, ar_f, W_b, al_b):
    grid = (N // BLK,)
    mat_spec = pl.BlockSpec((H, H), lambda i: (0, 0))
    vec_spec = pl.BlockSpec((H, 1), lambda i: (0, 0))
    z_spec = pl.BlockSpec((BLK, QW), lambda i: (i, 0))
    col_spec = pl.BlockSpec((BLK, 1), lambda i: (i, 0))
    zs = jax.ShapeDtypeStruct((N, QW), jnp.float32)
    cs = jax.ShapeDtypeStruct((N, 1), jnp.float32)
    return pl.pallas_call(
        _user_dense_body,
        grid=grid,
        in_specs=[pl.BlockSpec((BLK, H), lambda i: (i, 0)), mat_spec,
                  pl.BlockSpec((1, H), lambda i: (0, 0)),
                  mat_spec, vec_spec, vec_spec, mat_spec, vec_spec],
        out_specs=(z_spec,) * (2 * NQ) + (col_spec,) * 3,
        out_shape=(zs,) * (2 * NQ) + (cs,) * 3,
    )(feat_user, W_emb_user, b_emb_user.reshape(1, H), W_f,
      al_f.reshape(H, 1), ar_f.reshape(H, 1), W_b, al_b.reshape(H, 1))


def _dense_item(feat_item, W_emb_item, b_emb_item, W_b, ar_b):
    grid = (N // BLK,)
    mat_spec = pl.BlockSpec((H, H), lambda i: (0, 0))
    return pl.pallas_call(
        _item_dense_body,
        grid=grid,
        in_specs=[pl.BlockSpec((BLK, H), lambda i: (i, 0)), mat_spec,
                  pl.BlockSpec((1, H), lambda i: (0, 0)), mat_spec,
                  pl.BlockSpec((H, 1), lambda i: (0, 0))],
        out_specs=pl.BlockSpec((BLK, 1), lambda i: (i, 0)),
        out_shape=jax.ShapeDtypeStruct((N, 1), jnp.float32),
    )(feat_item, W_emb_item, b_emb_item.reshape(1, H), W_b, ar_b.reshape(H, 1))


_MESH = plsc.VectorSubcoreMesh(core_axis_name="c", subcore_axis_name="s")


@functools.partial(
    pl.kernel, mesh=_MESH,
    compiler_params=pltpu.CompilerParams(needs_layout_passes=False,
                                         use_tc_tiling_on_sc=False),
    out_type=jax.ShapeDtypeStruct((NQ * 2 * N, QB), jnp.float32),
    scratch_types=[
        pltpu.VMEM((N,), jnp.float32),
        pltpu.VMEM((N,), jnp.float32),
        pltpu.VMEM((EPT,), jnp.int32),
        pltpu.VMEM((NB, BATCH), jnp.int32),
        pltpu.VMEM((EPT,), jnp.float32),
        pltpu.VMEM((BATCH, QW), jnp.float32),
        pltpu.VMEM((BATCH, QW), jnp.float32),
        pltpu.VMEM((BATCH, QW), jnp.float32),
        pltpu.VMEM((BATCH, QB), jnp.float32),
        pltpu.VMEM_SHARED((NP, QW), jnp.float32),
        pltpu.SemaphoreType.DMA,
        pltpu.SemaphoreType.DMA,
    ],
)
def _edge_sc(z_hbm, el_hbm, er_hbm, src_hbm, dst3_hbm, out_hbm,
             el_v, er_v, srcp_v, dst2d, ex_v, rowbuf, rowbuf2, zbuf,
             outbuf, num_sh, sem, sem2):
    c = lax.axis_index("c")
    s = lax.axis_index("s")

    pltpu.sync_copy(el_hbm.at[pl.ds(c * N, N)], el_v)
    pltpu.sync_copy(er_hbm.at[pl.ds(c * N, N)], er_v)
    pltpu.sync_copy(src_hbm.at[pl.ds(c * E + s * EPT, EPT)], srcp_v)
    pltpu.sync_copy(dst3_hbm.at[c * NS + s], dst2d)

    def _zrow(r, _):
        for v in range(QW // 16):
            zbuf[r, pl.ds(v * 16, 16)] = jnp.zeros((16,), jnp.float32)
        return 0
    lax.fori_loop(0, BATCH, _zrow, 0)

    def _phA(j, _):
        for u in range(BATCH // 16):
            sl = pl.ds(j * BATCH + u * 16, 16)
            srcv = srcp_v[sl] - c * N
            dv = dst2d[j, pl.ds(u * 16, 16)]
            e = plsc.load_gather(el_v, [srcv]) + plsc.load_gather(er_v, [dv])
            e = jnp.where(e > 0, e, 0.2 * e)
            ex_v[sl] = jnp.exp(e)
        return 0
    lax.fori_loop(0, NB, _phA, 0)

    def _pass(p, _):
        @pl.when(p > 0)
        def _advance():
            def _off(i, _):
                sl = pl.ds(i * 16, 16)
                srcp_v[sl] = srcp_v[sl] + 2 * N
                return 0
            lax.fori_loop(0, EPT // 16, _off, 0)

        def _zero(b, _):
            pltpu.sync_copy(zbuf, num_sh.at[pl.ds(s * STRIPE + b * BATCH, BATCH)])
            return 0
        lax.fori_loop(0, STRIPE // BATCH, _zero, 0)
        plsc.subcore_barrier()

        def _gather(j, buf, sm):
            return pltpu.async_copy(z_hbm.at[srcp_v.at[pl.ds(j * BATCH, BATCH)]],
                                    buf, sm)

        def _proc(j, buf):
            def _scale(g, _):
                exv = ex_v[pl.ds(j * BATCH + g * 16, 16)]
                for k in range(16):
                    exs = exv[k]
                    r = g * 16 + k
                    for v in range(QW // 16):
                        sl2 = pl.ds(v * 16, 16)
                        buf[r, sl2] = buf[r, sl2] * exs
                return 0
            lax.fori_loop(0, BATCH // 16, _scale, 0)
            pltpu.sync_copy(buf, num_sh.at[dst2d.at[j]], add=True)

        def _gwait(buf, sm):
            pltpu.make_async_copy(z_hbm.at[srcp_v.at[pl.ds(0, BATCH)]],
                                  buf, sm).wait()

        _gather(0, rowbuf, sem)

        def _phB(j2, _):
            j = 2 * j2
            _gwait(rowbuf, sem)
            _gather(j + 1, rowbuf2, sem2)
            _proc(j, rowbuf)
            _gwait(rowbuf2, sem2)
            _gather(j + 2, rowbuf, sem)
            _proc(j + 1, rowbuf2)
            return 0
        lax.fori_loop(0, NB // 2, _phB, 0)
        _gwait(rowbuf, sem)
        _proc(NB - 1, rowbuf)
        plsc.subcore_barrier()

        def _fin(b, _):
            row0 = s * STRIPE + b * BATCH

            @pl.when(row0 < N)
            def _():
                pltpu.sync_copy(num_sh.at[pl.ds(row0, BATCH)], rowbuf)

                def _grp(g, _):
                    ridx = lax.iota(jnp.int32, 16) + g * 16
                    cidx = jnp.full((16,), QB, jnp.int32)
                    den = plsc.load_gather(rowbuf, [ridx, cidx])
                    invv = jnp.ones((16,), jnp.float32) / (den + 1e-9)
                    for k in range(16):
                        inv = invv[k]
                        r = g * 16 + k
                        for v in range(QB // 16):
                            sl2 = pl.ds(v * 16, 16)
                            outbuf[r, sl2] = jnp.maximum(
                                rowbuf[r, sl2] * inv, 0.0)
                    return 0
                lax.fori_loop(0, BATCH // 16, _grp, 0)
                pltpu.sync_copy(
                    outbuf, out_hbm.at[pl.ds(p * 2 * N + c * N + row0, BATCH)])
            return 0
        lax.fori_loop(0, STRIPE // BATCH, _fin, 0)
        return 0
    lax.fori_loop(0, NQ, _pass, 0)


def kernel(feat_user, feat_item, edge_follows, edge_buys,
           W_emb_user, b_emb_user, W_emb_item, b_emb_item,
           W_follows, al_follows, ar_follows, W_buys, al_buys, ar_buys):
    edge_follows = edge_follows.astype(jnp.int32)
    edge_buys = edge_buys.astype(jnp.int32)
    du = _dense_user(feat_user, W_emb_user, b_emb_user,
                     W_follows, al_follows, ar_follows, W_buys, al_buys)
    qf = du[:NQ]
    qb = du[NQ:2 * NQ]
    el_f, er_f, el_b = du[2 * NQ], du[2 * NQ + 1], du[2 * NQ + 2]
    er_b = _dense_item(feat_item, W_emb_item, b_emb_item, W_buys, ar_buys)

    z_all = jnp.concatenate(
        [blk for p in range(NQ) for blk in (qf[p], qb[p])], axis=0)
    el_all = jnp.concatenate([el_f[:, 0], el_b[:, 0]])
    er_all = jnp.concatenate([er_f[:, 0], er_b[:, 0]])
    src_all = jnp.concatenate([edge_follows[0], edge_buys[0] + N])
    dst3 = jnp.concatenate(
        [edge_follows[1].reshape(NS, NB, BATCH),
         edge_buys[1].reshape(NS, NB, BATCH)], axis=0)

    out = _edge_sc(z_all, el_all, er_all, src_all, dst3)
    out_user = jnp.concatenate(
        [out[p * 2 * N:p * 2 * N + N] for p in range(NQ)], axis=1)
    out_item = jnp.concatenate(
        [out[p * 2 * N + N:(p + 1) * 2 * N] for p in range(NQ)], axis=1)
    return out_user, out_item
